# baseline (speedup 1.0000x reference)
import math

import jax
import jax.numpy as jnp
from jax.experimental import pallas as pl
from jax.experimental.pallas import tpu as pltpu

_Z_COEF = 0.001
_AUX_COEF = 0.01
_CAP_FACTOR = 1.0
_MIN_CAP = 4


def _router_body(x_ref, wt_ref, disp_ref, comb_ref, probs_ref, aux_ref,
                 cnt_ref, psum_ref, z_ref):
    i = pl.program_id(0)
    nblk = pl.num_programs(0)
    TN = x_ref.shape[0]
    E = wt_ref.shape[1]
    C = comb_ref.shape[2]
    N = TN * nblk

    @pl.when(i == 0)
    def _init():
        cnt_ref[...] = jnp.zeros_like(cnt_ref)
        psum_ref[...] = jnp.zeros_like(psum_ref)
        z_ref[0, 0] = 0.0

    logits = jnp.dot(x_ref[...], wt_ref[...],
                     preferred_element_type=jnp.float32)
    m = jnp.max(logits, axis=1, keepdims=True)
    ex = jnp.exp(logits - m)
    s = jnp.sum(ex, axis=1, keepdims=True)
    probs = ex / s
    probs_ref[...] = probs
    lse = m + jnp.log(s)
    z_ref[0, 0] += jnp.sum(lse * lse)

    eio = jax.lax.broadcasted_iota(jnp.int32, (TN, E), 1)
    is_max = logits == m
    idx = jnp.min(jnp.where(is_max, eio, E), axis=1, keepdims=True)
    rw = jnp.max(probs, axis=1, keepdims=True)
    ohe = (eio == idx).astype(jnp.float32)

    r_i = jax.lax.broadcasted_iota(jnp.int32, (TN, TN), 0)
    c_i = jax.lax.broadcasted_iota(jnp.int32, (TN, TN), 1)
    tri = (r_i >= c_i).astype(jnp.float32)
    incl = jnp.dot(tri, ohe, preferred_element_type=jnp.float32)

    cnt = cnt_ref[...]
    pos = jnp.sum((incl + cnt) * ohe, axis=1, keepdims=True) - 1.0
    cnt_ref[...] = cnt + jnp.sum(ohe, axis=0, keepdims=True)
    psum_ref[...] += jnp.sum(probs, axis=0, keepdims=True)

    posi = pos.astype(jnp.int32)
    posk = jnp.where(posi < C, posi, -1)
    eio3 = jax.lax.broadcasted_iota(jnp.int32, (TN, E, C), 1)
    cio3 = jax.lax.broadcasted_iota(jnp.int32, (TN, E, C), 2)
    hit = (eio3 == idx[:, :, None]) & (cio3 == posk[:, :, None])
    comb_ref[...] = jnp.where(hit, rw[:, :, None], 0.0)
    disp_ref[...] = hit

    @pl.when(i == nblk - 1)
    def _fin():
        fi_pi = jnp.sum(cnt_ref[...] * psum_ref[...]) / (N * N)
        aux_ref[0, 0] = (_AUX_COEF * E * fi_pi
                         + _Z_COEF * (z_ref[0, 0] / N))


def kernel(x, W):
    B, T, D = x.shape
    N = B * T
    E = W.shape[0]
    C = max(int(math.ceil(_CAP_FACTOR * N / E)), _MIN_CAP)
    TN = 256
    nblk = N // TN

    xf = x.reshape(N, D)
    wt = W.T

    disp, comb, probs, aux = pl.pallas_call(
        _router_body,
        grid=(nblk,),
        in_specs=[
            pl.BlockSpec((TN, D), lambda i: (i, 0)),
            pl.BlockSpec((D, E), lambda i: (0, 0)),
        ],
        out_specs=[
            pl.BlockSpec((TN, E, C), lambda i: (i, 0, 0)),
            pl.BlockSpec((TN, E, C), lambda i: (i, 0, 0)),
            pl.BlockSpec((TN, E), lambda i: (i, 0)),
            pl.BlockSpec(memory_space=pltpu.SMEM),
        ],
        out_shape=[
            jax.ShapeDtypeStruct((N, E, C), jnp.bool_),
            jax.ShapeDtypeStruct((N, E, C), jnp.float32),
            jax.ShapeDtypeStruct((N, E), jnp.float32),
            jax.ShapeDtypeStruct((1, 1), jnp.float32),
        ],
        scratch_shapes=[
            pltpu.VMEM((1, E), jnp.float32),
            pltpu.VMEM((1, E), jnp.float32),
            pltpu.SMEM((1, 1), jnp.float32),
        ],
        compiler_params=pltpu.CompilerParams(
            dimension_semantics=("arbitrary",),
        ),
    )(xf, wt)

    return disp, comb, aux[0, 0], probs


# speedup vs baseline: 3.6374x; 3.6374x over previous
import math

import jax
import jax.numpy as jnp
from jax.experimental import pallas as pl
from jax.experimental.pallas import tpu as pltpu

_Z_COEF = 0.001
_AUX_COEF = 0.01
_CAP_FACTOR = 1.0
_MIN_CAP = 4


def _router_body(x_ref, wt_ref, disp_ref, comb_ref, probs_ref, aux_ref,
                 cnt_ref, psum_ref, z_ref):
    i = pl.program_id(0)
    nblk = pl.num_programs(0)
    TN = x_ref.shape[0]
    E = wt_ref.shape[1]
    C = comb_ref.shape[1]
    N = TN * nblk

    @pl.when(i == 0)
    def _init():
        cnt_ref[...] = jnp.zeros_like(cnt_ref)
        psum_ref[...] = jnp.zeros_like(psum_ref)
        z_ref[0, 0] = 0.0

    logits = jnp.dot(x_ref[...], wt_ref[...],
                     preferred_element_type=jnp.float32)
    lt = logits.T
    m = jnp.max(lt, axis=0, keepdims=True)
    ex = jnp.exp(lt - m)
    s = jnp.sum(ex, axis=0, keepdims=True)
    probs = ex / s
    probs_ref[...] = probs
    lse = m + jnp.log(s)
    z_ref[0, 0] += jnp.sum(lse * lse)

    eio = jax.lax.broadcasted_iota(jnp.int32, (E, TN), 0)
    idx = jnp.min(jnp.where(lt == m, eio, E), axis=0, keepdims=True)
    rw = jnp.max(probs, axis=0, keepdims=True)
    ohe = (eio == idx).astype(jnp.float32)

    r_i = jax.lax.broadcasted_iota(jnp.int32, (TN, TN), 0)
    c_i = jax.lax.broadcasted_iota(jnp.int32, (TN, TN), 1)
    tri = (r_i <= c_i).astype(jnp.float32)
    incl = jnp.dot(ohe, tri, preferred_element_type=jnp.float32)

    cnt = cnt_ref[...]
    pos = jnp.sum((incl + cnt) * ohe, axis=0, keepdims=True) - 1.0
    cnt_ref[...] = cnt + jnp.sum(ohe, axis=1, keepdims=True)
    psum_ref[...] += jnp.sum(probs, axis=1, keepdims=True)

    posi = pos.astype(jnp.int32)
    posk = jnp.where(posi < C, posi, -1)
    eio3 = jax.lax.broadcasted_iota(jnp.int32, (E, C, TN), 0)
    cio3 = jax.lax.broadcasted_iota(jnp.int32, (E, C, TN), 1)
    hit = (eio3 == idx[:, None, :]) & (cio3 == posk[:, None, :])
    comb_ref[...] = jnp.where(hit, rw[:, None, :], 0.0)
    disp_ref[...] = hit.astype(jnp.int8)

    @pl.when(i == nblk - 1)
    def _fin():
        fi_pi = jnp.sum(cnt_ref[...] * psum_ref[...]) / (N * N)
        aux_ref[0, 0] = (_AUX_COEF * E * fi_pi
                         + _Z_COEF * (z_ref[0, 0] / N))


def kernel(x, W):
    B, T, D = x.shape
    N = B * T
    E = W.shape[0]
    C = max(int(math.ceil(_CAP_FACTOR * N / E)), _MIN_CAP)
    TN = 256
    nblk = N // TN

    xf = x.reshape(N, D)
    wt = W.T

    disp_t, comb_t, probs_t, aux = pl.pallas_call(
        _router_body,
        grid=(nblk,),
        in_specs=[
            pl.BlockSpec((TN, D), lambda i: (i, 0)),
            pl.BlockSpec((D, E), lambda i: (0, 0)),
        ],
        out_specs=[
            pl.BlockSpec((E, C, TN), lambda i: (0, 0, i)),
            pl.BlockSpec((E, C, TN), lambda i: (0, 0, i)),
            pl.BlockSpec((E, TN), lambda i: (0, i)),
            pl.BlockSpec(memory_space=pltpu.SMEM),
        ],
        out_shape=[
            jax.ShapeDtypeStruct((E, C, N), jnp.int8),
            jax.ShapeDtypeStruct((E, C, N), jnp.float32),
            jax.ShapeDtypeStruct((E, N), jnp.float32),
            jax.ShapeDtypeStruct((1, 1), jnp.float32),
        ],
        scratch_shapes=[
            pltpu.VMEM((E, 1), jnp.float32),
            pltpu.VMEM((E, 1), jnp.float32),
            pltpu.SMEM((1, 1), jnp.float32),
        ],
        compiler_params=pltpu.CompilerParams(
            dimension_semantics=("arbitrary",),
        ),
    )(xf, wt)

    dispatch_mask = jnp.transpose(disp_t, (2, 0, 1)).astype(jnp.bool_)
    combine_weights = jnp.transpose(comb_t, (2, 0, 1))
    return dispatch_mask, combine_weights, aux[0, 0], probs_t.T


# speedup vs baseline: 4.1479x; 1.1404x over previous
import math

import jax
import jax.numpy as jnp
from jax.experimental import pallas as pl
from jax.experimental.pallas import tpu as pltpu

_Z_COEF = 0.001
_AUX_COEF = 0.01
_CAP_FACTOR = 1.0
_MIN_CAP = 4


def _router_body(x_ref, wt_ref, comb_ref, probs_ref, tgt_ref, aux_ref,
                 cnt_ref, psum_ref, z_ref):
    i = pl.program_id(0)
    nblk = pl.num_programs(0)
    TN = x_ref.shape[0]
    E = wt_ref.shape[1]
    C = comb_ref.shape[1]
    N = TN * nblk

    @pl.when(i == 0)
    def _init():
        cnt_ref[...] = jnp.zeros_like(cnt_ref)
        psum_ref[...] = jnp.zeros_like(psum_ref)
        z_ref[0, 0] = 0.0

    logits = jnp.dot(x_ref[...], wt_ref[...],
                     preferred_element_type=jnp.float32)
    lt = logits.T
    m = jnp.max(lt, axis=0, keepdims=True)
    ex = jnp.exp(lt - m)
    s = jnp.sum(ex, axis=0, keepdims=True)
    probs = ex / s
    probs_ref[...] = probs
    lse = m + jnp.log(s)
    z_ref[0, 0] += jnp.sum(lse * lse)

    eio = jax.lax.broadcasted_iota(jnp.int32, (E, TN), 0)
    idx = jnp.min(jnp.where(lt == m, eio, E), axis=0, keepdims=True)
    rw = jnp.max(probs, axis=0, keepdims=True)
    ohe = (eio == idx).astype(jnp.float32)

    r_i = jax.lax.broadcasted_iota(jnp.int32, (TN, TN), 0)
    c_i = jax.lax.broadcasted_iota(jnp.int32, (TN, TN), 1)
    tri = (r_i <= c_i).astype(jnp.float32)
    incl = jnp.dot(ohe, tri, preferred_element_type=jnp.float32)

    cnt = cnt_ref[...]
    pos = jnp.sum((incl + cnt) * ohe, axis=0, keepdims=True) - 1.0
    cnt_ref[...] = cnt + jnp.sum(ohe, axis=1, keepdims=True)
    psum_ref[...] += jnp.sum(probs, axis=1, keepdims=True)

    posi = pos.astype(jnp.int32)
    posk = jnp.where(posi < C, posi, -1)
    eio3 = jax.lax.broadcasted_iota(jnp.int32, (E, C, TN), 0)
    cio3 = jax.lax.broadcasted_iota(jnp.int32, (E, C, TN), 1)
    hit = (eio3 == idx[:, None, :]) & (cio3 == posk[:, None, :])
    comb_ref[...] = jnp.where(hit, rw[:, None, :], 0.0)
    tgt_ref[...] = jnp.where(posk >= 0, idx * C + posk, -1)

    @pl.when(i == nblk - 1)
    def _fin():
        fi_pi = jnp.sum(cnt_ref[...] * psum_ref[...]) / (N * N)
        aux_ref[0, 0] = (_AUX_COEF * E * fi_pi
                         + _Z_COEF * (z_ref[0, 0] / N))


def kernel(x, W):
    B, T, D = x.shape
    N = B * T
    E = W.shape[0]
    C = max(int(math.ceil(_CAP_FACTOR * N / E)), _MIN_CAP)
    TN = 256
    nblk = N // TN

    xf = x.reshape(N, D)
    wt = W.T

    comb_t, probs_t, tgt, aux = pl.pallas_call(
        _router_body,
        grid=(nblk,),
        in_specs=[
            pl.BlockSpec((TN, D), lambda i: (i, 0)),
            pl.BlockSpec((D, E), lambda i: (0, 0)),
        ],
        out_specs=[
            pl.BlockSpec((E, C, TN), lambda i: (0, 0, i)),
            pl.BlockSpec((E, TN), lambda i: (0, i)),
            pl.BlockSpec((1, TN), lambda i: (0, i)),
            pl.BlockSpec(memory_space=pltpu.SMEM),
        ],
        out_shape=[
            jax.ShapeDtypeStruct((E, C, N), jnp.float32),
            jax.ShapeDtypeStruct((E, N), jnp.float32),
            jax.ShapeDtypeStruct((1, N), jnp.int32),
            jax.ShapeDtypeStruct((1, 1), jnp.float32),
        ],
        scratch_shapes=[
            pltpu.VMEM((E, 1), jnp.float32),
            pltpu.VMEM((E, 1), jnp.float32),
            pltpu.SMEM((1, 1), jnp.float32),
        ],
        compiler_params=pltpu.CompilerParams(
            dimension_semantics=("arbitrary",),
        ),
    )(xf, wt)

    ecgrid = jnp.arange(E * C, dtype=jnp.int32).reshape(E, C)
    dispatch_mask = tgt.reshape(N)[:, None, None] == ecgrid[None]
    combine_weights = jnp.transpose(comb_t, (2, 0, 1))
    return dispatch_mask, combine_weights, aux[0, 0], probs_t.T


# speedup vs baseline: 4.3538x; 1.0496x over previous
import math

import jax
import jax.numpy as jnp
from jax.experimental import pallas as pl
from jax.experimental.pallas import tpu as pltpu

_Z_COEF = 0.001
_AUX_COEF = 0.01
_CAP_FACTOR = 1.0
_MIN_CAP = 4


def _router_body(x_ref, wt_ref, comb_ref, probs_ref, tgt_ref, aux_ref,
                 cnt_ref, psum_ref, z_ref):
    i = pl.program_id(0)
    nblk = pl.num_programs(0)
    TN = x_ref.shape[0]
    E = wt_ref.shape[1]
    C = comb_ref.shape[1]
    N = TN * nblk

    @pl.when(i == 0)
    def _init():
        cnt_ref[...] = jnp.zeros_like(cnt_ref)
        psum_ref[...] = jnp.zeros_like(psum_ref)
        z_ref[0, 0] = 0.0

    logits = jnp.dot(x_ref[...], wt_ref[...],
                     preferred_element_type=jnp.float32)
    lt = logits.T
    m = jnp.max(lt, axis=0, keepdims=True)
    ex = jnp.exp(lt - m)
    s = jnp.sum(ex, axis=0, keepdims=True)
    probs = ex / s
    probs_ref[...] = probs
    lse = m + jnp.log(s)
    z_ref[0, 0] += jnp.sum(lse * lse)

    eio = jax.lax.broadcasted_iota(jnp.int32, (E, TN), 0)
    idx = jnp.min(jnp.where(lt == m, eio, E), axis=0, keepdims=True)
    rw = jnp.max(probs, axis=0, keepdims=True)
    ohe = (eio == idx).astype(jnp.float32)

    r_i = jax.lax.broadcasted_iota(jnp.int32, (TN, TN), 0)
    c_i = jax.lax.broadcasted_iota(jnp.int32, (TN, TN), 1)
    tri = (r_i <= c_i).astype(jnp.float32)
    incl = jnp.dot(ohe, tri, preferred_element_type=jnp.float32)

    cnt = cnt_ref[...]
    pos = jnp.sum((incl + cnt) * ohe, axis=0, keepdims=True) - 1.0
    cnt_ref[...] = cnt + jnp.sum(ohe, axis=1, keepdims=True)
    psum_ref[...] += jnp.sum(probs, axis=1, keepdims=True)

    posi = pos.astype(jnp.int32)
    posk = jnp.where(posi < C, posi, -1)
    eio3 = jax.lax.broadcasted_iota(jnp.int32, (E, C, TN), 0)
    cio3 = jax.lax.broadcasted_iota(jnp.int32, (E, C, TN), 1)
    hit = (eio3 == idx[:, None, :]) & (cio3 == posk[:, None, :])
    comb_ref[...] = jnp.where(hit, rw[:, None, :], 0.0)
    tgt_ref[...] = jnp.where(posk >= 0, idx * C + posk, -1)

    @pl.when(i == nblk - 1)
    def _fin():
        fi_pi = jnp.sum(cnt_ref[...] * psum_ref[...]) / (N * N)
        aux_ref[0, 0] = (_AUX_COEF * E * fi_pi
                         + _Z_COEF * (z_ref[0, 0] / N))


def kernel(x, W):
    B, T, D = x.shape
    N = B * T
    E = W.shape[0]
    C = max(int(math.ceil(_CAP_FACTOR * N / E)), _MIN_CAP)
    TN = 512
    nblk = N // TN

    xf = x.reshape(N, D)
    wt = W.T

    comb_t, probs_t, tgt, aux = pl.pallas_call(
        _router_body,
        grid=(nblk,),
        in_specs=[
            pl.BlockSpec((TN, D), lambda i: (i, 0)),
            pl.BlockSpec((D, E), lambda i: (0, 0)),
        ],
        out_specs=[
            pl.BlockSpec((E, C, TN), lambda i: (0, 0, i)),
            pl.BlockSpec((E, TN), lambda i: (0, i)),
            pl.BlockSpec((1, TN), lambda i: (0, i)),
            pl.BlockSpec(memory_space=pltpu.SMEM),
        ],
        out_shape=[
            jax.ShapeDtypeStruct((E, C, N), jnp.float32),
            jax.ShapeDtypeStruct((E, N), jnp.float32),
            jax.ShapeDtypeStruct((1, N), jnp.int32),
            jax.ShapeDtypeStruct((1, 1), jnp.float32),
        ],
        scratch_shapes=[
            pltpu.VMEM((E, 1), jnp.float32),
            pltpu.VMEM((E, 1), jnp.float32),
            pltpu.SMEM((1, 1), jnp.float32),
        ],
        compiler_params=pltpu.CompilerParams(
            dimension_semantics=("arbitrary",),
        ),
    )(xf, wt)

    ecgrid = jnp.arange(E * C, dtype=jnp.int32).reshape(E, C)
    dispatch_mask = tgt.reshape(N)[:, None, None] == ecgrid[None]
    combine_weights = jnp.transpose(comb_t, (2, 0, 1))
    return dispatch_mask, combine_weights, aux[0, 0], probs_t.T
